# baseline (speedup 1.0000x reference)
import functools

import jax
import jax.numpy as jnp
from jax import lax
from jax.experimental import pallas as pl
from jax.experimental.pallas import tpu as pltpu
from jax.experimental.pallas import tpu_sc as plsc

_D = 64
_CH = 128
_NC = 2
_NS = 16
_NW = _NC * _NS
_BB = 2048


def _sc_gather_body(uid_hbm, iid_hbm, ut_hbm, it_hbm, u_out, i_out,
                    uidx_v, iidx_v, urows_v, irows_v, sem, *, nch):
    wid = lax.axis_index("s") * _NC + lax.axis_index("c")
    base = wid * nch
    pltpu.sync_copy(uid_hbm.at[pl.ds(base, nch)], uidx_v)
    pltpu.sync_copy(iid_hbm.at[pl.ds(base, nch)], iidx_v)
    cps = []
    for j in range(nch):
        cps.append(pltpu.async_copy(ut_hbm.at[uidx_v.at[j]], urows_v.at[j], sem))
        cps.append(pltpu.async_copy(it_hbm.at[iidx_v.at[j]], irows_v.at[j], sem))
    for cp in cps:
        cp.wait()
    pltpu.sync_copy(urows_v, u_out.at[pl.ds(base, nch)])
    pltpu.sync_copy(irows_v, i_out.at[pl.ds(base, nch)])


def _sc_gather(uids2, iids2, user_table, item_table):
    nb = uids2.shape[0]
    nch = nb // _NW
    row_t = jax.ShapeDtypeStruct((nb, _CH, _D), jnp.float32)
    k = pl.kernel(
        functools.partial(_sc_gather_body, nch=nch),
        mesh=plsc.VectorSubcoreMesh(core_axis_name="c", subcore_axis_name="s"),
        compiler_params=pltpu.CompilerParams(use_tc_tiling_on_sc=False),
        out_type=[row_t, row_t],
        scratch_types=[
            pltpu.VMEM((nch, _CH), jnp.int32),
            pltpu.VMEM((nch, _CH), jnp.int32),
            pltpu.VMEM((nch, _CH, _D), jnp.float32),
            pltpu.VMEM((nch, _CH, _D), jnp.float32),
            pltpu.SemaphoreType.DMA,
        ],
    )
    return k(uids2, iids2, user_table, item_table)


def _mlp_body(u_ref, i_ref, w1u_ref, w1i_ref, b1_ref, w2_ref, b2_ref,
              w3_ref, b3_ref, w4t_ref, b4_ref, o_ref):
    h = jnp.dot(u_ref[...], w1u_ref[...], preferred_element_type=jnp.float32)
    h = h + jnp.dot(i_ref[...], w1i_ref[...], preferred_element_type=jnp.float32)
    h = jnp.maximum(h + b1_ref[...], 0.0)
    h = jnp.maximum(
        jnp.dot(h, w2_ref[...], preferred_element_type=jnp.float32) + b2_ref[...], 0.0)
    h = jnp.maximum(
        jnp.dot(h, w3_ref[...], preferred_element_type=jnp.float32) + b3_ref[...], 0.0)
    o_ref[...] = jnp.sum(h * w4t_ref[...], axis=1, keepdims=True) + b4_ref[...]


def kernel(user_ids, item_ids, user_table, item_table,
           W1, b1, W2, b2, W3, b3, W4, b4):
    batch = user_ids.shape[0]
    nb = batch // _CH
    uids2 = user_ids.astype(jnp.int32).reshape(nb, _CH)
    iids2 = item_ids.astype(jnp.int32).reshape(nb, _CH)

    u3, i3 = _sc_gather(uids2, iids2, user_table, item_table)
    u = u3.reshape(batch, _D)
    i = i3.reshape(batch, _D)

    w1u = W1[:_D]
    w1i = W1[_D:]
    b1r = b1.reshape(1, -1)
    b2r = b2.reshape(1, -1)
    b3r = b3.reshape(1, -1)
    w4t = W4.reshape(1, -1)
    b4r = b4.reshape(1, 1)

    out = pl.pallas_call(
        _mlp_body,
        grid=(batch // _BB,),
        in_specs=[
            pl.BlockSpec((_BB, _D), lambda b: (b, 0)),
            pl.BlockSpec((_BB, _D), lambda b: (b, 0)),
            pl.BlockSpec(w1u.shape, lambda b: (0, 0)),
            pl.BlockSpec(w1i.shape, lambda b: (0, 0)),
            pl.BlockSpec(b1r.shape, lambda b: (0, 0)),
            pl.BlockSpec(W2.shape, lambda b: (0, 0)),
            pl.BlockSpec(b2r.shape, lambda b: (0, 0)),
            pl.BlockSpec(W3.shape, lambda b: (0, 0)),
            pl.BlockSpec(b3r.shape, lambda b: (0, 0)),
            pl.BlockSpec(w4t.shape, lambda b: (0, 0)),
            pl.BlockSpec(b4r.shape, lambda b: (0, 0)),
        ],
        out_specs=pl.BlockSpec((_BB, 1), lambda b: (b, 0)),
        out_shape=jax.ShapeDtypeStruct((batch, 1), jnp.float32),
        compiler_params=pltpu.CompilerParams(
            dimension_semantics=("arbitrary",)),
    )(u, i, w1u, w1i, b1r, W2, b2r, W3, b3r, w4t, b4r)
    return out[:, 0]


# speedup vs baseline: 1.5697x; 1.5697x over previous
import functools

import jax
import jax.numpy as jnp
from jax import lax
from jax.experimental import pallas as pl
from jax.experimental.pallas import tpu as pltpu
from jax.experimental.pallas import tpu_sc as plsc

_D = 64
_NC = 2
_NS = 16
_NW = _NC * _NS
_L = 16
_BB = 2048


def _sc_gather_body(uid_hbm, iid_hbm, ut_hbm, it_hbm, u_out, i_out,
                    uidx_v, iidx_v, urows_v, irows_v, sem, *, bpw, rpp):
    wid = lax.axis_index("s") * _NC + lax.axis_index("c")
    base = wid * bpw
    pltpu.sync_copy(uid_hbm.at[pl.ds(base, bpw)], uidx_v)
    pltpu.sync_copy(iid_hbm.at[pl.ds(base, bpw)], iidx_v)

    for p in range(bpw // rpp):
        def group(g, _):
            uv = uidx_v[pl.ds(p * rpp + g * _L, _L)]
            iv = iidx_v[pl.ds(p * rpp + g * _L, _L)]
            for j in range(_L):
                pltpu.async_copy(ut_hbm.at[pl.ds(uv[j], 1)],
                                 urows_v.at[pl.ds(g * _L + j, 1)], sem)
                pltpu.async_copy(it_hbm.at[pl.ds(iv[j], 1)],
                                 irows_v.at[pl.ds(g * _L + j, 1)], sem)
            return ()

        lax.fori_loop(0, rpp // _L, group, (), unroll=False)
        pltpu.make_async_copy(ut_hbm.at[pl.ds(0, rpp)], urows_v, sem).wait()
        pltpu.make_async_copy(it_hbm.at[pl.ds(0, rpp)], irows_v, sem).wait()
        pltpu.sync_copy(urows_v, u_out.at[pl.ds(base + p * rpp, rpp)])
        pltpu.sync_copy(irows_v, i_out.at[pl.ds(base + p * rpp, rpp)])


def _sc_gather(uids, iids, user_table, item_table):
    batch = uids.shape[0]
    bpw = batch // _NW
    rpp = min(bpw, 256)
    row_t = jax.ShapeDtypeStruct((batch, _D), jnp.float32)
    k = pl.kernel(
        functools.partial(_sc_gather_body, bpw=bpw, rpp=rpp),
        mesh=plsc.VectorSubcoreMesh(core_axis_name="c", subcore_axis_name="s"),
        compiler_params=pltpu.CompilerParams(use_tc_tiling_on_sc=True),
        out_type=[row_t, row_t],
        scratch_types=[
            pltpu.VMEM((bpw,), jnp.int32),
            pltpu.VMEM((bpw,), jnp.int32),
            pltpu.VMEM((rpp, _D), jnp.float32),
            pltpu.VMEM((rpp, _D), jnp.float32),
            pltpu.SemaphoreType.DMA,
        ],
    )
    return k(uids, iids, user_table, item_table)


def _mlp_body(u_ref, i_ref, w1u_ref, w1i_ref, b1_ref, w2_ref, b2_ref,
              w3_ref, b3_ref, w4t_ref, b4_ref, o_ref):
    h = jnp.dot(u_ref[...], w1u_ref[...], preferred_element_type=jnp.float32)
    h = h + jnp.dot(i_ref[...], w1i_ref[...], preferred_element_type=jnp.float32)
    h = jnp.maximum(h + b1_ref[...], 0.0)
    h = jnp.maximum(
        jnp.dot(h, w2_ref[...], preferred_element_type=jnp.float32) + b2_ref[...], 0.0)
    h = jnp.maximum(
        jnp.dot(h, w3_ref[...], preferred_element_type=jnp.float32) + b3_ref[...], 0.0)
    o_ref[...] = jnp.sum(h * w4t_ref[...], axis=1, keepdims=True) + b4_ref[...]


def kernel(user_ids, item_ids, user_table, item_table,
           W1, b1, W2, b2, W3, b3, W4, b4):
    batch = user_ids.shape[0]
    uids = user_ids.astype(jnp.int32)
    iids = item_ids.astype(jnp.int32)

    u, i = _sc_gather(uids, iids, user_table, item_table)

    w1u = W1[:_D]
    w1i = W1[_D:]
    b1r = b1.reshape(1, -1)
    b2r = b2.reshape(1, -1)
    b3r = b3.reshape(1, -1)
    w4t = W4.reshape(1, -1)
    b4r = b4.reshape(1, 1)

    out = pl.pallas_call(
        _mlp_body,
        grid=(batch // _BB,),
        in_specs=[
            pl.BlockSpec((_BB, _D), lambda b: (b, 0)),
            pl.BlockSpec((_BB, _D), lambda b: (b, 0)),
            pl.BlockSpec(w1u.shape, lambda b: (0, 0)),
            pl.BlockSpec(w1i.shape, lambda b: (0, 0)),
            pl.BlockSpec(b1r.shape, lambda b: (0, 0)),
            pl.BlockSpec(W2.shape, lambda b: (0, 0)),
            pl.BlockSpec(b2r.shape, lambda b: (0, 0)),
            pl.BlockSpec(W3.shape, lambda b: (0, 0)),
            pl.BlockSpec(b3r.shape, lambda b: (0, 0)),
            pl.BlockSpec(w4t.shape, lambda b: (0, 0)),
            pl.BlockSpec(b4r.shape, lambda b: (0, 0)),
        ],
        out_specs=pl.BlockSpec((_BB, 1), lambda b: (b, 0)),
        out_shape=jax.ShapeDtypeStruct((batch, 1), jnp.float32),
        compiler_params=pltpu.CompilerParams(
            dimension_semantics=("arbitrary",)),
    )(u, i, w1u, w1i, b1r, W2, b2r, W3, b3r, w4t, b4r)
    return out[:, 0]
